# baseline (speedup 1.0000x reference)
import functools

import jax
import jax.numpy as jnp
from jax import lax
from jax.experimental import pallas as pl
from jax.experimental.pallas import tpu as pltpu
from jax.experimental.pallas import tpu_sc as plsc

EPS2 = 2.0 ** 0.5
N_NODES = 10000
N_EDGES = 320000
D = 128
SQRT_D = float(D) ** 0.5

NC = 2
NS = 16
NW = NC * NS

P = 10240
RPT = P // NS

CH = 125
EPW = N_EDGES // NW
NCHUNK = EPW // CH
GRP = 8
NGRP = NCHUNK // GRP

DW = D

BLK = P // 8

@functools.cache
def _sc_mesh():
    return plsc.VectorSubcoreMesh(
        core_axis_name="c", subcore_axis_name="s",
        num_cores=NC, num_subcores=NS)


def _deg_body(ei_hbm, out_hbm, sidx, ones_v, acc, ssem, isem):
    cid = lax.axis_index("c")
    sid = lax.axis_index("s")
    wid = sid * NC + cid
    c0 = wid * NCHUNK
    pltpu.async_copy(ei_hbm.at[1, pl.ds(c0, NCHUNK), :], sidx, isem)

    def _fill_zero(i, _):
        r = i // (DW // 16)
        col = lax.rem(i, DW // 16) * 16
        ones_v[r, pl.ds(col, 16)] = jnp.zeros((16,), jnp.float32)
        return 0
    lax.fori_loop(0, (128 * DW) // 16, _fill_zero, 0, unroll=8)
    for k in range(RPT // 128):
        pltpu.sync_copy(ones_v, acc.at[pl.ds(sid * RPT + k * 128, 128)])

    def _fill_ones(i, _):
        r = i // (DW // 16)
        col = lax.rem(i, DW // 16) * 16
        ones_v[r, pl.ds(col, 16)] = jnp.ones((16,), jnp.float32)
        return 0
    lax.fori_loop(0, (128 * DW) // 16, _fill_ones, 0, unroll=8)

    pltpu.make_async_copy(ei_hbm.at[1, pl.ds(c0, NCHUNK), :], sidx, isem).wait()
    plsc.subcore_barrier()

    def _fire(j, _):
        pltpu.async_copy(ones_v.at[pl.ds(0, CH), :],
                         acc.at[sidx.at[j]], ssem, add=True).wait()
        return 0
    lax.fori_loop(0, NCHUNK, _fire, 0)

    plsc.subcore_barrier()
    pltpu.sync_copy(acc.at[pl.ds(sid * RPT, RPT)],
                    out_hbm.at[cid, pl.ds(sid * RPT, RPT), :])


@functools.cache
def _deg_call():
    return pl.kernel(
        _deg_body,
        out_type=jax.ShapeDtypeStruct((NC, P, DW), jnp.float32),
        mesh=_sc_mesh(),
        scratch_types=[
            pltpu.VMEM((NCHUNK, CH), jnp.int32),
            pltpu.VMEM((128, DW), jnp.float32),
            pltpu.VMEM_SHARED((P, DW), jnp.float32),
            pltpu.SemaphoreType.DMA,
            pltpu.SemaphoreType.DMA,
        ],
    )


def _mp_body(h_hbm, ei_hbm, out_hbm, sidx, didx, rows, acc,
             gsem0, gsem1, ssem0, ssem1, isem):
    cid = lax.axis_index("c")
    sid = lax.axis_index("s")
    wid = sid * NC + cid
    c0 = wid * NCHUNK
    pltpu.async_copy(ei_hbm.at[1, pl.ds(c0, NCHUNK), :], sidx, isem)

    def _fill_zero(i, _):
        b = i // (128 * (D // 16))
        r = lax.rem(i, 128 * (D // 16)) // (D // 16)
        col = lax.rem(i, D // 16) * 16
        rows[b, r, pl.ds(col, 16)] = jnp.zeros((16,), jnp.float32)
        return 0
    lax.fori_loop(0, 2 * 128 * (D // 16), _fill_zero, 0, unroll=8)
    for k in range(RPT // 128):
        pltpu.sync_copy(rows.at[0], acc.at[pl.ds(sid * RPT + k * 128, 128)])

    pltpu.make_async_copy(ei_hbm.at[1, pl.ds(c0, NCHUNK), :], sidx, isem).wait()

    pltpu.async_copy(ei_hbm.at[0, pl.ds(c0, GRP), :], didx.at[0], isem)
    pltpu.make_async_copy(ei_hbm.at[0, pl.ds(c0, GRP), :], didx.at[0], isem).wait()
    pltpu.async_copy(ei_hbm.at[0, pl.ds(c0 + GRP, GRP), :], didx.at[1], isem)
    plsc.subcore_barrier()

    pltpu.async_copy(h_hbm.at[sidx.at[0]], rows.at[0, pl.ds(0, CH), :], gsem0)

    def _pair(k, _):
        a = 2 * k
        g = a // GRP
        gslot = lax.rem(g, 2)
        sub = lax.rem(a, GRP)

        pltpu.make_async_copy(h_hbm.at[sidx.at[a]],
                              rows.at[0, pl.ds(0, CH), :], gsem0).wait()

        @pl.when(k > 0)
        def _():
            pltpu.make_async_copy(rows.at[1, pl.ds(0, CH), :],
                                  acc.at[didx.at[0, 0]], ssem1).wait()

        @pl.when((sub == 0) & (g > 0))
        def _():
            pltpu.make_async_copy(ei_hbm.at[0, pl.ds(c0 + g * GRP, GRP), :],
                                  didx.at[gslot], isem).wait()

            @pl.when(g + 1 < NGRP)
            def _():
                pltpu.async_copy(ei_hbm.at[0, pl.ds(c0 + (g + 1) * GRP, GRP), :],
                                 didx.at[1 - gslot], isem)

        pltpu.async_copy(h_hbm.at[sidx.at[a + 1]],
                         rows.at[1, pl.ds(0, CH), :], gsem1)
        pltpu.async_copy(rows.at[0, pl.ds(0, CH), :],
                         acc.at[didx.at[gslot, sub]], ssem0, add=True)

        pltpu.make_async_copy(h_hbm.at[sidx.at[a + 1]],
                              rows.at[1, pl.ds(0, CH), :], gsem1).wait()
        pltpu.make_async_copy(rows.at[0, pl.ds(0, CH), :],
                              acc.at[didx.at[0, 0]], ssem0).wait()

        @pl.when(a + 2 < NCHUNK)
        def _():
            pltpu.async_copy(h_hbm.at[sidx.at[a + 2]],
                             rows.at[0, pl.ds(0, CH), :], gsem0)

        pltpu.async_copy(rows.at[1, pl.ds(0, CH), :],
                         acc.at[didx.at[gslot, sub + 1]], ssem1, add=True)
        return 0
    lax.fori_loop(0, NCHUNK // 2, _pair, 0)
    pltpu.make_async_copy(rows.at[1, pl.ds(0, CH), :],
                          acc.at[didx.at[0, 0]], ssem1).wait()

    plsc.subcore_barrier()
    pltpu.sync_copy(acc.at[pl.ds(sid * RPT, RPT)],
                    out_hbm.at[cid, pl.ds(sid * RPT, RPT), :])


@functools.cache
def _mp_call():
    return pl.kernel(
        _mp_body,
        out_type=jax.ShapeDtypeStruct((NC, P, D), jnp.float32),
        mesh=_sc_mesh(),
        scratch_types=[
            pltpu.VMEM((NCHUNK, CH), jnp.int32),
            pltpu.VMEM((2, GRP, CH), jnp.int32),
            pltpu.VMEM((2, 128, D), jnp.float32),
            pltpu.VMEM_SHARED((P, D), jnp.float32),
            pltpu.SemaphoreType.DMA,
            pltpu.SemaphoreType.DMA,
            pltpu.SemaphoreType.DMA,
            pltpu.SemaphoreType.DMA,
            pltpu.SemaphoreType.DMA,
        ],
    )


def _dg(a, w):
    return lax.dot_general(a, w, (((1,), (1,)), ((), ())),
                           preferred_element_type=jnp.float32)


def _first_tc(x_ref, w_ref, b_ref, degp_ref, h_ref, invn_ref):
    h = jnp.maximum(_dg(x_ref[...], w_ref[...]) + b_ref[...], 0.0)
    deg = degp_ref[0, :, 0:1] + degp_ref[1, :, 0:1]
    invn = 1.0 / (SQRT_D * (1.0 + EPS2 + deg))
    invn_b = jnp.broadcast_to(invn, h.shape)
    h_ref[...] = h * invn_b
    invn_ref[...] = invn_b


def _combine_tc(h_ref, p_ref, w_ref, b_ref, invn_ref, o_ref):
    t = (1.0 + EPS2) * h_ref[...] + p_ref[0] + p_ref[1]
    o_ref[...] = jnp.maximum(_dg(t, w_ref[...]) + b_ref[...], 0.0) * invn_ref[...]


def _tail_tc(h_ref, p_ref, fw0, fb0, fw1, fb1, pw0, pb0, pw1, pb1, ow, ob,
             o_ref):
    t = (1.0 + EPS2) * h_ref[...] + p_ref[0] + p_ref[1]
    h4 = _dg(jnp.maximum(t, 0.0), fw0[...]) + fb0[...] + _dg(t, pw0[...]) + pb0[...]
    h5 = _dg(jnp.maximum(h4, 0.0), fw1[...]) + fb1[...] + _dg(t, pw1[...]) + pb1[...]
    o_ref[...] = _dg(h5, ow[...]) + ob[...]


_row_spec = pl.BlockSpec((BLK, D), lambda i: (i, 0))
_w_spec = pl.BlockSpec((D, D), lambda i: (0, 0))
_b_spec = pl.BlockSpec((1, D), lambda i: (0, 0))
_p_spec = pl.BlockSpec((NC, BLK, D), lambda i: (0, i, 0))

_first_call = pl.pallas_call(
    _first_tc,
    grid=(P // BLK,),
    in_specs=[_row_spec, _w_spec, _b_spec,
              pl.BlockSpec((NC, BLK, DW), lambda i: (0, i, 0))],
    out_specs=[_row_spec, _row_spec],
    out_shape=[jax.ShapeDtypeStruct((P, D), jnp.float32),
               jax.ShapeDtypeStruct((P, D), jnp.float32)],
)

_combine_call = pl.pallas_call(
    _combine_tc,
    grid=(P // BLK,),
    in_specs=[_row_spec, _p_spec, _w_spec, _b_spec, _row_spec],
    out_specs=_row_spec,
    out_shape=jax.ShapeDtypeStruct((P, D), jnp.float32),
)

_tail_call = pl.pallas_call(
    _tail_tc,
    grid=(P // BLK,),
    in_specs=[_row_spec, _p_spec] + [_w_spec, _b_spec] * 5,
    out_specs=_row_spec,
    out_shape=jax.ShapeDtypeStruct((P, D), jnp.float32),
)


def kernel(x, edge_index, mp_w0, mp_b0, mp_w1, mp_b1, mp_w2, mp_b2,
           fc_w0, fc_b0, fc_w1, fc_b1, pj_w0, pj_b0, pj_w1, pj_b1,
           out_w, out_b):
    xp = jnp.pad(x, ((0, P - N_NODES), (0, 0)))
    ei3 = edge_index.reshape(2, N_EDGES // CH, CH)

    degp = _deg_call()(ei3)
    h, invn = _first_call(xp, mp_w0, mp_b0.reshape(1, D), degp)
    for w, b in ((mp_w1, mp_b1), (mp_w2, mp_b2)):
        part = _mp_call()(h, ei3)
        h = _combine_call(h, part, w, b.reshape(1, D), invn)
    part = _mp_call()(h, ei3)
    outp = _tail_call(h, part,
                      fc_w0, fc_b0.reshape(1, D), fc_w1, fc_b1.reshape(1, D),
                      pj_w0, pj_b0.reshape(1, D), pj_w1, pj_b1.reshape(1, D),
                      out_w, out_b.reshape(1, D))
    return outp[:N_NODES]


# speedup vs baseline: 1.0002x; 1.0002x over previous
import functools

import jax
import jax.numpy as jnp
from jax import lax
from jax.experimental import pallas as pl
from jax.experimental.pallas import tpu as pltpu
from jax.experimental.pallas import tpu_sc as plsc

EPS2 = 2.0 ** 0.5
N_NODES = 10000
N_EDGES = 320000
D = 128
SQRT_D = float(D) ** 0.5

NC = 2
NS = 16
NW = NC * NS

P = 10240
RPT = P // NS

CH = 125
EPW = N_EDGES // NW
NCHUNK = EPW // CH
GRP = 8
NGRP = NCHUNK // GRP

DW = 16

BLK = P // 8

@functools.cache
def _sc_mesh():
    return plsc.VectorSubcoreMesh(
        core_axis_name="c", subcore_axis_name="s",
        num_cores=NC, num_subcores=NS)


def _deg_body(ei_hbm, out_hbm, sidx, ones_v, acc, ssem, isem):
    cid = lax.axis_index("c")
    sid = lax.axis_index("s")
    wid = sid * NC + cid
    c0 = wid * NCHUNK
    pltpu.async_copy(ei_hbm.at[1, pl.ds(c0, NCHUNK), :], sidx, isem)

    def _fill_zero(i, _):
        r = i // (D // 16)
        col = lax.rem(i, D // 16) * 16
        ones_v[r, pl.ds(col, 16)] = jnp.zeros((16,), jnp.float32)
        return 0
    lax.fori_loop(0, (128 * D) // 16, _fill_zero, 0, unroll=8)
    for k in range(RPT // 128):
        pltpu.sync_copy(ones_v, acc.at[pl.ds(sid * RPT + k * 128, 128)])

    def _fill_ones(i, _):
        r = i // (D // 16)
        col = lax.rem(i, D // 16) * 16
        ones_v[r, pl.ds(col, 16)] = jnp.ones((16,), jnp.float32)
        return 0
    lax.fori_loop(0, (128 * D) // 16, _fill_ones, 0, unroll=8)

    pltpu.make_async_copy(ei_hbm.at[1, pl.ds(c0, NCHUNK), :], sidx, isem).wait()
    plsc.subcore_barrier()

    def _fire(j, _):
        pltpu.async_copy(ones_v.at[pl.ds(0, CH), :],
                         acc.at[sidx.at[j]], ssem, add=True)

        @pl.when(j >= 3)
        def _():
            pltpu.make_async_copy(ones_v.at[pl.ds(0, CH), :],
                                  acc.at[sidx.at[0]], ssem).wait()
        return 0
    lax.fori_loop(0, NCHUNK, _fire, 0)
    for _ in range(3):
        pltpu.make_async_copy(ones_v.at[pl.ds(0, CH), :],
                              acc.at[sidx.at[0]], ssem).wait()

    plsc.subcore_barrier()
    pltpu.sync_copy(acc.at[pl.ds(sid * RPT, RPT)],
                    out_hbm.at[cid, pl.ds(sid * RPT, RPT), :])


@functools.cache
def _deg_call():
    return pl.kernel(
        _deg_body,
        out_type=jax.ShapeDtypeStruct((NC, P, D), jnp.float32),
        mesh=_sc_mesh(),
        scratch_types=[
            pltpu.VMEM((NCHUNK, CH), jnp.int32),
            pltpu.VMEM((128, D), jnp.float32),
            pltpu.VMEM_SHARED((P, D), jnp.float32),
            pltpu.SemaphoreType.DMA,
            pltpu.SemaphoreType.DMA,
        ],
    )


def _mp_body(h_hbm, ei_hbm, out_hbm, sidx, didx, rows, acc,
             gsem0, gsem1, ssem0, ssem1, isem):
    cid = lax.axis_index("c")
    sid = lax.axis_index("s")
    wid = sid * NC + cid
    c0 = wid * NCHUNK
    pltpu.async_copy(ei_hbm.at[1, pl.ds(c0, NCHUNK), :], sidx, isem)

    def _fill_zero(i, _):
        b = i // (128 * (D // 16))
        r = lax.rem(i, 128 * (D // 16)) // (D // 16)
        col = lax.rem(i, D // 16) * 16
        rows[b, r, pl.ds(col, 16)] = jnp.zeros((16,), jnp.float32)
        return 0
    lax.fori_loop(0, 2 * 128 * (D // 16), _fill_zero, 0, unroll=8)
    for k in range(RPT // 128):
        pltpu.sync_copy(rows.at[0], acc.at[pl.ds(sid * RPT + k * 128, 128)])

    pltpu.make_async_copy(ei_hbm.at[1, pl.ds(c0, NCHUNK), :], sidx, isem).wait()

    pltpu.async_copy(ei_hbm.at[0, pl.ds(c0, GRP), :], didx.at[0], isem)
    pltpu.make_async_copy(ei_hbm.at[0, pl.ds(c0, GRP), :], didx.at[0], isem).wait()
    pltpu.async_copy(ei_hbm.at[0, pl.ds(c0 + GRP, GRP), :], didx.at[1], isem)
    plsc.subcore_barrier()

    pltpu.async_copy(h_hbm.at[sidx.at[0]], rows.at[0, pl.ds(0, CH), :], gsem0)

    def _pair(k, _):
        a = 2 * k
        g = a // GRP
        gslot = lax.rem(g, 2)
        sub = lax.rem(a, GRP)

        pltpu.make_async_copy(h_hbm.at[sidx.at[a]],
                              rows.at[0, pl.ds(0, CH), :], gsem0).wait()

        @pl.when(k > 0)
        def _():
            pltpu.make_async_copy(rows.at[1, pl.ds(0, CH), :],
                                  acc.at[didx.at[0, 0]], ssem1).wait()

        @pl.when((sub == 0) & (g > 0))
        def _():
            pltpu.make_async_copy(ei_hbm.at[0, pl.ds(c0 + g * GRP, GRP), :],
                                  didx.at[gslot], isem).wait()

            @pl.when(g + 1 < NGRP)
            def _():
                pltpu.async_copy(ei_hbm.at[0, pl.ds(c0 + (g + 1) * GRP, GRP), :],
                                 didx.at[1 - gslot], isem)

        pltpu.async_copy(h_hbm.at[sidx.at[a + 1]],
                         rows.at[1, pl.ds(0, CH), :], gsem1)
        pltpu.async_copy(rows.at[0, pl.ds(0, CH), :],
                         acc.at[didx.at[gslot, sub]], ssem0, add=True)

        pltpu.make_async_copy(h_hbm.at[sidx.at[a + 1]],
                              rows.at[1, pl.ds(0, CH), :], gsem1).wait()
        pltpu.make_async_copy(rows.at[0, pl.ds(0, CH), :],
                              acc.at[didx.at[0, 0]], ssem0).wait()

        @pl.when(a + 2 < NCHUNK)
        def _():
            pltpu.async_copy(h_hbm.at[sidx.at[a + 2]],
                             rows.at[0, pl.ds(0, CH), :], gsem0)

        pltpu.async_copy(rows.at[1, pl.ds(0, CH), :],
                         acc.at[didx.at[gslot, sub + 1]], ssem1, add=True)
        return 0
    lax.fori_loop(0, NCHUNK // 2, _pair, 0)
    pltpu.make_async_copy(rows.at[1, pl.ds(0, CH), :],
                          acc.at[didx.at[0, 0]], ssem1).wait()

    plsc.subcore_barrier()
    pltpu.sync_copy(acc.at[pl.ds(sid * RPT, RPT)],
                    out_hbm.at[cid, pl.ds(sid * RPT, RPT), :])


@functools.cache
def _mp_call():
    return pl.kernel(
        _mp_body,
        out_type=jax.ShapeDtypeStruct((NC, P, D), jnp.float32),
        mesh=_sc_mesh(),
        scratch_types=[
            pltpu.VMEM((NCHUNK, CH), jnp.int32),
            pltpu.VMEM((2, GRP, CH), jnp.int32),
            pltpu.VMEM((2, 128, D), jnp.float32),
            pltpu.VMEM_SHARED((P, D), jnp.float32),
            pltpu.SemaphoreType.DMA,
            pltpu.SemaphoreType.DMA,
            pltpu.SemaphoreType.DMA,
            pltpu.SemaphoreType.DMA,
            pltpu.SemaphoreType.DMA,
        ],
    )


def _dg(a, w):
    return lax.dot_general(a, w, (((1,), (1,)), ((), ())),
                           preferred_element_type=jnp.float32)


def _first_tc(x_ref, w_ref, b_ref, degp_ref, h_ref, invn_ref):
    h = jnp.maximum(_dg(x_ref[...], w_ref[...]) + b_ref[...], 0.0)
    deg = degp_ref[0, :, 0:1] + degp_ref[1, :, 0:1]
    invn = 1.0 / (SQRT_D * (1.0 + EPS2 + deg))
    invn_b = jnp.broadcast_to(invn, h.shape)
    h_ref[...] = h * invn_b
    invn_ref[...] = invn_b


def _combine_tc(h_ref, p_ref, w_ref, b_ref, invn_ref, o_ref):
    t = (1.0 + EPS2) * h_ref[...] + p_ref[0] + p_ref[1]
    o_ref[...] = jnp.maximum(_dg(t, w_ref[...]) + b_ref[...], 0.0) * invn_ref[...]


def _tail_tc(h_ref, p_ref, fw0, fb0, fw1, fb1, pw0, pb0, pw1, pb1, ow, ob,
             o_ref):
    t = (1.0 + EPS2) * h_ref[...] + p_ref[0] + p_ref[1]
    h4 = _dg(jnp.maximum(t, 0.0), fw0[...]) + fb0[...] + _dg(t, pw0[...]) + pb0[...]
    h5 = _dg(jnp.maximum(h4, 0.0), fw1[...]) + fb1[...] + _dg(t, pw1[...]) + pb1[...]
    o_ref[...] = _dg(h5, ow[...]) + ob[...]


_row_spec = pl.BlockSpec((BLK, D), lambda i: (i, 0))
_w_spec = pl.BlockSpec((D, D), lambda i: (0, 0))
_b_spec = pl.BlockSpec((1, D), lambda i: (0, 0))
_p_spec = pl.BlockSpec((NC, BLK, D), lambda i: (0, i, 0))

_first_call = pl.pallas_call(
    _first_tc,
    grid=(P // BLK,),
    in_specs=[_row_spec, _w_spec, _b_spec,
              pl.BlockSpec((NC, BLK, D), lambda i: (0, i, 0))],
    out_specs=[_row_spec, _row_spec],
    out_shape=[jax.ShapeDtypeStruct((P, D), jnp.float32),
               jax.ShapeDtypeStruct((P, D), jnp.float32)],
)

_combine_call = pl.pallas_call(
    _combine_tc,
    grid=(P // BLK,),
    in_specs=[_row_spec, _p_spec, _w_spec, _b_spec, _row_spec],
    out_specs=_row_spec,
    out_shape=jax.ShapeDtypeStruct((P, D), jnp.float32),
)

_tail_call = pl.pallas_call(
    _tail_tc,
    grid=(P // BLK,),
    in_specs=[_row_spec, _p_spec] + [_w_spec, _b_spec] * 5,
    out_specs=_row_spec,
    out_shape=jax.ShapeDtypeStruct((P, D), jnp.float32),
)


def kernel(x, edge_index, mp_w0, mp_b0, mp_w1, mp_b1, mp_w2, mp_b2,
           fc_w0, fc_b0, fc_w1, fc_b1, pj_w0, pj_b0, pj_w1, pj_b1,
           out_w, out_b):
    xp = jnp.pad(x, ((0, P - N_NODES), (0, 0)))
    ei3 = edge_index.reshape(2, N_EDGES // CH, CH)

    degp = _deg_call()(ei3)
    h, invn = _first_call(xp, mp_w0, mp_b0.reshape(1, D), degp)
    for w, b in ((mp_w1, mp_b1), (mp_w2, mp_b2)):
        part = _mp_call()(h, ei3)
        h = _combine_call(h, part, w, b.reshape(1, D), invn)
    part = _mp_call()(h, ei3)
    outp = _tail_call(h, part,
                      fc_w0, fc_b0.reshape(1, D), fc_w1, fc_b1.reshape(1, D),
                      pj_w0, pj_b0.reshape(1, D), pj_w1, pj_b1.reshape(1, D),
                      out_w, out_b.reshape(1, D))
    return outp[:N_NODES]


# speedup vs baseline: 1.0020x; 1.0018x over previous
import functools

import jax
import jax.numpy as jnp
from jax import lax
from jax.experimental import pallas as pl
from jax.experimental.pallas import tpu as pltpu
from jax.experimental.pallas import tpu_sc as plsc

EPS2 = 2.0 ** 0.5
N_NODES = 10000
N_EDGES = 320000
D = 128
SQRT_D = float(D) ** 0.5

NC = 2
NS = 16
NW = NC * NS

P = 10240
RPT = P // NS

CH = 125
EPW = N_EDGES // NW
NCHUNK = EPW // CH
GRP = 8
NGRP = NCHUNK // GRP

DW = 16

BLK = P // 8

@functools.cache
def _sc_mesh():
    return plsc.VectorSubcoreMesh(
        core_axis_name="c", subcore_axis_name="s",
        num_cores=NC, num_subcores=NS)


def _deg_body(ei_hbm, out_hbm, sidx, ones_v, acc, ssem, isem):
    cid = lax.axis_index("c")
    sid = lax.axis_index("s")
    wid = sid * NC + cid
    c0 = wid * NCHUNK
    pltpu.async_copy(ei_hbm.at[1, pl.ds(c0, NCHUNK), :], sidx, isem)

    def _fill_zero(i, _):
        r = i // (D // 16)
        col = lax.rem(i, D // 16) * 16
        ones_v[r, pl.ds(col, 16)] = jnp.zeros((16,), jnp.float32)
        return 0
    lax.fori_loop(0, (128 * D) // 16, _fill_zero, 0, unroll=8)
    for k in range(RPT // 128):
        pltpu.sync_copy(ones_v, acc.at[pl.ds(sid * RPT + k * 128, 128)])

    def _fill_ones(i, _):
        r = i // (D // 16)
        col = lax.rem(i, D // 16) * 16
        ones_v[r, pl.ds(col, 16)] = jnp.ones((16,), jnp.float32)
        return 0
    lax.fori_loop(0, (128 * D) // 16, _fill_ones, 0, unroll=8)

    pltpu.make_async_copy(ei_hbm.at[1, pl.ds(c0, NCHUNK), :], sidx, isem).wait()
    plsc.subcore_barrier()

    def _fire(j, _):
        pltpu.async_copy(ones_v.at[pl.ds(0, CH), :],
                         acc.at[sidx.at[j]], ssem, add=True)

        @pl.when(j >= 3)
        def _():
            pltpu.make_async_copy(ones_v.at[pl.ds(0, CH), :],
                                  acc.at[sidx.at[0]], ssem).wait()
        return 0
    lax.fori_loop(0, NCHUNK, _fire, 0)
    for _ in range(3):
        pltpu.make_async_copy(ones_v.at[pl.ds(0, CH), :],
                              acc.at[sidx.at[0]], ssem).wait()

    plsc.subcore_barrier()
    pltpu.sync_copy(acc.at[pl.ds(sid * RPT, RPT)],
                    out_hbm.at[cid, pl.ds(sid * RPT, RPT), :])


@functools.cache
def _deg_call():
    return pl.kernel(
        _deg_body,
        out_type=jax.ShapeDtypeStruct((NC, P, D), jnp.float32),
        mesh=_sc_mesh(),
        scratch_types=[
            pltpu.VMEM((NCHUNK, CH), jnp.int32),
            pltpu.VMEM((128, D), jnp.float32),
            pltpu.VMEM_SHARED((P, D), jnp.float32),
            pltpu.SemaphoreType.DMA,
            pltpu.SemaphoreType.DMA,
        ],
    )


def _mp_body(h_hbm, ei_hbm, out_hbm, sidx, didx, rows, acc,
             gsem0, gsem1, ssem0, ssem1, isem):
    cid = lax.axis_index("c")
    sid = lax.axis_index("s")
    wid = sid * NC + cid
    c0 = wid * NCHUNK
    pltpu.async_copy(ei_hbm.at[1, pl.ds(c0, NCHUNK), :], sidx, isem)

    def _fill_zero(i, _):
        b = i // (128 * (D // 16))
        r = lax.rem(i, 128 * (D // 16)) // (D // 16)
        col = lax.rem(i, D // 16) * 16
        rows[b, r, pl.ds(col, 16)] = jnp.zeros((16,), jnp.float32)
        return 0
    lax.fori_loop(0, 2 * 128 * (D // 16), _fill_zero, 0, unroll=8)
    for k in range(RPT // 128):
        pltpu.sync_copy(rows.at[0], acc.at[pl.ds(sid * RPT + k * 128, 128)])

    pltpu.make_async_copy(ei_hbm.at[1, pl.ds(c0, NCHUNK), :], sidx, isem).wait()

    pltpu.async_copy(ei_hbm.at[0, pl.ds(c0, GRP), :], didx.at[0], isem)
    pltpu.make_async_copy(ei_hbm.at[0, pl.ds(c0, GRP), :], didx.at[0], isem).wait()
    pltpu.async_copy(ei_hbm.at[0, pl.ds(c0 + GRP, GRP), :], didx.at[1], isem)
    plsc.subcore_barrier()

    pltpu.async_copy(h_hbm.at[sidx.at[0]], rows.at[0, pl.ds(0, CH), :], gsem0)

    def _pair(k, _):
        a = 2 * k
        g = a // GRP
        gslot = lax.rem(g, 2)
        sub = lax.rem(a, GRP)

        pltpu.make_async_copy(h_hbm.at[sidx.at[a]],
                              rows.at[0, pl.ds(0, CH), :], gsem0).wait()

        @pl.when(k > 0)
        def _():
            pltpu.make_async_copy(rows.at[1, pl.ds(0, CH), :],
                                  acc.at[didx.at[0, 0]], ssem1).wait()

        @pl.when((sub == 0) & (g > 0))
        def _():
            pltpu.make_async_copy(ei_hbm.at[0, pl.ds(c0 + g * GRP, GRP), :],
                                  didx.at[gslot], isem).wait()

            @pl.when(g + 1 < NGRP)
            def _():
                pltpu.async_copy(ei_hbm.at[0, pl.ds(c0 + (g + 1) * GRP, GRP), :],
                                 didx.at[1 - gslot], isem)

        pltpu.async_copy(h_hbm.at[sidx.at[a + 1]],
                         rows.at[1, pl.ds(0, CH), :], gsem1)
        pltpu.async_copy(rows.at[0, pl.ds(0, CH), :],
                         acc.at[didx.at[gslot, sub]], ssem0, add=True)

        pltpu.make_async_copy(h_hbm.at[sidx.at[a + 1]],
                              rows.at[1, pl.ds(0, CH), :], gsem1).wait()
        pltpu.make_async_copy(rows.at[0, pl.ds(0, CH), :],
                              acc.at[didx.at[0, 0]], ssem0).wait()

        @pl.when(a + 2 < NCHUNK)
        def _():
            pltpu.async_copy(h_hbm.at[sidx.at[a + 2]],
                             rows.at[0, pl.ds(0, CH), :], gsem0)

        pltpu.async_copy(rows.at[1, pl.ds(0, CH), :],
                         acc.at[didx.at[gslot, sub + 1]], ssem1, add=True)
        return 0
    lax.fori_loop(0, NCHUNK // 2, _pair, 0)
    pltpu.make_async_copy(rows.at[1, pl.ds(0, CH), :],
                          acc.at[didx.at[0, 0]], ssem1).wait()

    plsc.subcore_barrier()
    pltpu.sync_copy(acc.at[pl.ds(sid * RPT, RPT)],
                    out_hbm.at[cid, pl.ds(sid * RPT, RPT), :])


@functools.cache
def _mp_call():
    return pl.kernel(
        _mp_body,
        out_type=jax.ShapeDtypeStruct((NC, P, D), jnp.float32),
        mesh=_sc_mesh(),
        scratch_types=[
            pltpu.VMEM((NCHUNK, CH), jnp.int32),
            pltpu.VMEM((2, GRP, CH), jnp.int32),
            pltpu.VMEM((2, 128, D), jnp.float32),
            pltpu.VMEM_SHARED((P, D), jnp.float32),
            pltpu.SemaphoreType.DMA,
            pltpu.SemaphoreType.DMA,
            pltpu.SemaphoreType.DMA,
            pltpu.SemaphoreType.DMA,
            pltpu.SemaphoreType.DMA,
        ],
    )


def _dg(a, w):
    return lax.dot_general(a, w, (((1,), (1,)), ((), ())),
                           preferred_element_type=jnp.float32)


def _u_tc(x_ref, w_ref, b_ref, u_ref):
    u_ref[...] = jnp.maximum(_dg(x_ref[...], w_ref[...]) + b_ref[...], 0.0)


def _scale_tc(u_ref, degp_ref, h_ref, invn_ref):
    deg = degp_ref[0, :, 0:1] + degp_ref[1, :, 0:1]
    invn = 1.0 / (SQRT_D * (1.0 + EPS2 + deg))
    invn_b = jnp.broadcast_to(invn, u_ref.shape)
    h_ref[...] = u_ref[...] * invn_b
    invn_ref[...] = invn_b


def _combine_tc(h_ref, p_ref, w_ref, b_ref, invn_ref, o_ref):
    t = (1.0 + EPS2) * h_ref[...] + p_ref[0] + p_ref[1]
    o_ref[...] = jnp.maximum(_dg(t, w_ref[...]) + b_ref[...], 0.0) * invn_ref[...]


def _tail_tc(h_ref, p_ref, fw0, fb0, fw1, fb1, pw0, pb0, pw1, pb1, ow, ob,
             o_ref):
    t = (1.0 + EPS2) * h_ref[...] + p_ref[0] + p_ref[1]
    h4 = _dg(jnp.maximum(t, 0.0), fw0[...]) + fb0[...] + _dg(t, pw0[...]) + pb0[...]
    h5 = _dg(jnp.maximum(h4, 0.0), fw1[...]) + fb1[...] + _dg(t, pw1[...]) + pb1[...]
    o_ref[...] = _dg(h5, ow[...]) + ob[...]


_row_spec = pl.BlockSpec((BLK, D), lambda i: (i, 0))
_w_spec = pl.BlockSpec((D, D), lambda i: (0, 0))
_b_spec = pl.BlockSpec((1, D), lambda i: (0, 0))
_p_spec = pl.BlockSpec((NC, BLK, D), lambda i: (0, i, 0))

_u_call = pl.pallas_call(
    _u_tc,
    grid=(P // BLK,),
    in_specs=[_row_spec, _w_spec, _b_spec],
    out_specs=_row_spec,
    out_shape=jax.ShapeDtypeStruct((P, D), jnp.float32),
)

_scale_call = pl.pallas_call(
    _scale_tc,
    grid=(P // BLK,),
    in_specs=[_row_spec, pl.BlockSpec((NC, BLK, D), lambda i: (0, i, 0))],
    out_specs=[_row_spec, _row_spec],
    out_shape=[jax.ShapeDtypeStruct((P, D), jnp.float32),
               jax.ShapeDtypeStruct((P, D), jnp.float32)],
)

_combine_call = pl.pallas_call(
    _combine_tc,
    grid=(P // BLK,),
    in_specs=[_row_spec, _p_spec, _w_spec, _b_spec, _row_spec],
    out_specs=_row_spec,
    out_shape=jax.ShapeDtypeStruct((P, D), jnp.float32),
)

_tail_call = pl.pallas_call(
    _tail_tc,
    grid=(P // BLK,),
    in_specs=[_row_spec, _p_spec] + [_w_spec, _b_spec] * 5,
    out_specs=_row_spec,
    out_shape=jax.ShapeDtypeStruct((P, D), jnp.float32),
)


def kernel(x, edge_index, mp_w0, mp_b0, mp_w1, mp_b1, mp_w2, mp_b2,
           fc_w0, fc_b0, fc_w1, fc_b1, pj_w0, pj_b0, pj_w1, pj_b1,
           out_w, out_b):
    xp = jnp.pad(x, ((0, P - N_NODES), (0, 0)))
    ei3 = edge_index.reshape(2, N_EDGES // CH, CH)

    u = _u_call(xp, mp_w0, mp_b0.reshape(1, D))
    degp = _deg_call()(ei3)
    h, invn = _scale_call(u, degp)
    for w, b in ((mp_w1, mp_b1), (mp_w2, mp_b2)):
        part = _mp_call()(h, ei3)
        h = _combine_call(h, part, w, b.reshape(1, D), invn)
    part = _mp_call()(h, ei3)
    outp = _tail_call(h, part,
                      fc_w0, fc_b0.reshape(1, D), fc_w1, fc_b1.reshape(1, D),
                      pj_w0, pj_b0.reshape(1, D), pj_w1, pj_b1.reshape(1, D),
                      out_w, out_b.reshape(1, D))
    return outp[:N_NODES]


# speedup vs baseline: 1.0037x; 1.0017x over previous
import functools

import jax
import jax.numpy as jnp
from jax import lax
from jax.experimental import pallas as pl
from jax.experimental.pallas import tpu as pltpu
from jax.experimental.pallas import tpu_sc as plsc

EPS2 = 2.0 ** 0.5
N_NODES = 10000
N_EDGES = 320000
D = 128
SQRT_D = float(D) ** 0.5

NC = 2
NS = 16
NW = NC * NS

P = 10240
RPT = P // NS

CH = 125
EPW = N_EDGES // NW
NCHUNK = EPW // CH
GRP = 8
NGRP = NCHUNK // GRP

BLK = P // 8

@functools.cache
def _sc_mesh():
    return plsc.VectorSubcoreMesh(
        core_axis_name="c", subcore_axis_name="s",
        num_cores=NC, num_subcores=NS)


def _deg_body(ei_hbm, out_hbm, sidx, ones_v, acc, ssem, isem):
    cid = lax.axis_index("c")
    sid = lax.axis_index("s")
    wid = sid * NC + cid
    c0 = wid * NCHUNK
    pltpu.async_copy(ei_hbm.at[1, pl.ds(c0, NCHUNK), :], sidx, isem)

    def _fill_zero(i, _):
        r = i // (D // 16)
        col = lax.rem(i, D // 16) * 16
        ones_v[r, pl.ds(col, 16)] = jnp.zeros((16,), jnp.float32)
        return 0
    lax.fori_loop(0, (128 * D) // 16, _fill_zero, 0, unroll=8)
    for k in range(RPT // 128):
        pltpu.sync_copy(ones_v, acc.at[pl.ds(sid * RPT + k * 128, 128)])

    def _fill_ones(i, _):
        r = i // (D // 16)
        col = lax.rem(i, D // 16) * 16
        ones_v[r, pl.ds(col, 16)] = jnp.ones((16,), jnp.float32)
        return 0
    lax.fori_loop(0, (128 * D) // 16, _fill_ones, 0, unroll=8)

    pltpu.make_async_copy(ei_hbm.at[1, pl.ds(c0, NCHUNK), :], sidx, isem).wait()
    plsc.subcore_barrier()

    def _fire(j, _):
        pltpu.async_copy(ones_v.at[pl.ds(0, CH), :],
                         acc.at[sidx.at[j]], ssem, add=True)

        @pl.when(j >= 3)
        def _():
            pltpu.make_async_copy(ones_v.at[pl.ds(0, CH), :],
                                  acc.at[sidx.at[0]], ssem).wait()
        return 0
    lax.fori_loop(0, NCHUNK, _fire, 0)
    for _ in range(3):
        pltpu.make_async_copy(ones_v.at[pl.ds(0, CH), :],
                              acc.at[sidx.at[0]], ssem).wait()

    plsc.subcore_barrier()
    pltpu.sync_copy(acc.at[pl.ds(sid * RPT, RPT)],
                    out_hbm.at[cid, pl.ds(sid * RPT, RPT), :])


@functools.cache
def _deg_call():
    return pl.kernel(
        _deg_body,
        out_type=jax.ShapeDtypeStruct((NC, P, D), jnp.float32),
        mesh=_sc_mesh(),
        scratch_types=[
            pltpu.VMEM((NCHUNK, CH), jnp.int32),
            pltpu.VMEM((128, D), jnp.float32),
            pltpu.VMEM_SHARED((P, D), jnp.float32),
            pltpu.SemaphoreType.DMA,
            pltpu.SemaphoreType.DMA,
        ],
    )


def _mp_body(h_hbm, ei_hbm, out_hbm, sidx, didx, rows, acc,
             gsem0, gsem1, ssem0, ssem1, isem):
    cid = lax.axis_index("c")
    sid = lax.axis_index("s")
    wid = sid * NC + cid
    c0 = wid * NCHUNK
    pltpu.async_copy(ei_hbm.at[1, pl.ds(c0, NCHUNK), :], sidx, isem)

    def _fill_zero(i, _):
        b = i // (128 * (D // 16))
        r = lax.rem(i, 128 * (D // 16)) // (D // 16)
        col = lax.rem(i, D // 16) * 16
        rows[b, r, pl.ds(col, 16)] = jnp.zeros((16,), jnp.float32)
        return 0
    lax.fori_loop(0, 2 * 128 * (D // 16), _fill_zero, 0, unroll=8)
    for k in range(RPT // 128):
        pltpu.sync_copy(rows.at[0], acc.at[pl.ds(sid * RPT + k * 128, 128)])

    pltpu.make_async_copy(ei_hbm.at[1, pl.ds(c0, NCHUNK), :], sidx, isem).wait()

    pltpu.async_copy(ei_hbm.at[0, pl.ds(c0, GRP), :], didx.at[0], isem)
    pltpu.make_async_copy(ei_hbm.at[0, pl.ds(c0, GRP), :], didx.at[0], isem).wait()
    pltpu.async_copy(ei_hbm.at[0, pl.ds(c0 + GRP, GRP), :], didx.at[1], isem)
    plsc.subcore_barrier()

    pltpu.async_copy(h_hbm.at[sidx.at[0]], rows.at[0, pl.ds(0, CH), :], gsem0)

    def _pair(k, _):
        a = 2 * k
        g = a // GRP
        gslot = lax.rem(g, 2)
        sub = lax.rem(a, GRP)

        pltpu.make_async_copy(h_hbm.at[sidx.at[a]],
                              rows.at[0, pl.ds(0, CH), :], gsem0).wait()

        @pl.when(k > 0)
        def _():
            pltpu.make_async_copy(rows.at[1, pl.ds(0, CH), :],
                                  acc.at[didx.at[0, 0]], ssem1).wait()

        @pl.when((sub == 0) & (g > 0))
        def _():
            pltpu.make_async_copy(ei_hbm.at[0, pl.ds(c0 + g * GRP, GRP), :],
                                  didx.at[gslot], isem).wait()

            @pl.when(g + 1 < NGRP)
            def _():
                pltpu.async_copy(ei_hbm.at[0, pl.ds(c0 + (g + 1) * GRP, GRP), :],
                                 didx.at[1 - gslot], isem)

        pltpu.async_copy(h_hbm.at[sidx.at[a + 1]],
                         rows.at[1, pl.ds(0, CH), :], gsem1)
        pltpu.async_copy(rows.at[0, pl.ds(0, CH), :],
                         acc.at[didx.at[gslot, sub]], ssem0, add=True)

        pltpu.make_async_copy(h_hbm.at[sidx.at[a + 1]],
                              rows.at[1, pl.ds(0, CH), :], gsem1).wait()
        pltpu.make_async_copy(rows.at[0, pl.ds(0, CH), :],
                              acc.at[didx.at[0, 0]], ssem0).wait()

        @pl.when(a + 2 < NCHUNK)
        def _():
            pltpu.async_copy(h_hbm.at[sidx.at[a + 2]],
                             rows.at[0, pl.ds(0, CH), :], gsem0)

        pltpu.async_copy(rows.at[1, pl.ds(0, CH), :],
                         acc.at[didx.at[gslot, sub + 1]], ssem1, add=True)
        return 0
    lax.fori_loop(0, NCHUNK // 2, _pair, 0)
    pltpu.make_async_copy(rows.at[1, pl.ds(0, CH), :],
                          acc.at[didx.at[0, 0]], ssem1).wait()

    plsc.subcore_barrier()
    pltpu.sync_copy(acc.at[pl.ds(sid * RPT, RPT)],
                    out_hbm.at[cid, pl.ds(sid * RPT, RPT), :])


@functools.cache
def _mp_call():
    return pl.kernel(
        _mp_body,
        out_type=jax.ShapeDtypeStruct((NC, P, D), jnp.float32),
        mesh=_sc_mesh(),
        scratch_types=[
            pltpu.VMEM((NCHUNK, CH), jnp.int32),
            pltpu.VMEM((2, GRP, CH), jnp.int32),
            pltpu.VMEM((2, 128, D), jnp.float32),
            pltpu.VMEM_SHARED((P, D), jnp.float32),
            pltpu.SemaphoreType.DMA,
            pltpu.SemaphoreType.DMA,
            pltpu.SemaphoreType.DMA,
            pltpu.SemaphoreType.DMA,
            pltpu.SemaphoreType.DMA,
        ],
    )


def _dg(a, w):
    return lax.dot_general(a, w, (((1,), (1,)), ((), ())),
                           preferred_element_type=jnp.float32)


def _u_tc(x_ref, w_ref, b_ref, u_ref):
    u_ref[...] = jnp.maximum(_dg(x_ref[...], w_ref[...]) + b_ref[...], 0.0)


def _scale_tc(u_ref, degp_ref, h_ref, invn_ref):
    deg = degp_ref[0, :, 0:1] + degp_ref[1, :, 0:1]
    invn = 1.0 / (SQRT_D * (1.0 + EPS2 + deg))
    h_ref[...] = u_ref[...] * jnp.broadcast_to(invn, u_ref.shape)
    invn_ref[...] = invn


def _combine_tc(h_ref, p_ref, w_ref, b_ref, invn_ref, o_ref):
    t = (1.0 + EPS2) * h_ref[...] + p_ref[0] + p_ref[1]
    o_ref[...] = (jnp.maximum(_dg(t, w_ref[...]) + b_ref[...], 0.0)
                  * jnp.broadcast_to(invn_ref[...], t.shape))


def _tail_tc(h_ref, p_ref, fw0, fb0, fw1, fb1, pw0, pb0, pw1, pb1, ow, ob,
             o_ref):
    t = (1.0 + EPS2) * h_ref[...] + p_ref[0] + p_ref[1]
    h4 = _dg(jnp.maximum(t, 0.0), fw0[...]) + fb0[...] + _dg(t, pw0[...]) + pb0[...]
    h5 = _dg(jnp.maximum(h4, 0.0), fw1[...]) + fb1[...] + _dg(t, pw1[...]) + pb1[...]
    o_ref[...] = _dg(h5, ow[...]) + ob[...]


_row_spec = pl.BlockSpec((BLK, D), lambda i: (i, 0))
_w_spec = pl.BlockSpec((D, D), lambda i: (0, 0))
_b_spec = pl.BlockSpec((1, D), lambda i: (0, 0))
_p_spec = pl.BlockSpec((NC, BLK, D), lambda i: (0, i, 0))

_u_call = pl.pallas_call(
    _u_tc,
    grid=(P // BLK,),
    in_specs=[_row_spec, _w_spec, _b_spec],
    out_specs=_row_spec,
    out_shape=jax.ShapeDtypeStruct((P, D), jnp.float32),
)

_c_spec = pl.BlockSpec((BLK, 1), lambda i: (i, 0))

_scale_call = pl.pallas_call(
    _scale_tc,
    grid=(P // BLK,),
    in_specs=[_row_spec, _p_spec],
    out_specs=[_row_spec, _c_spec],
    out_shape=[jax.ShapeDtypeStruct((P, D), jnp.float32),
               jax.ShapeDtypeStruct((P, 1), jnp.float32)],
)

_combine_call = pl.pallas_call(
    _combine_tc,
    grid=(P // BLK,),
    in_specs=[_row_spec, _p_spec, _w_spec, _b_spec, _c_spec],
    out_specs=_row_spec,
    out_shape=jax.ShapeDtypeStruct((P, D), jnp.float32),
)

_tail_call = pl.pallas_call(
    _tail_tc,
    grid=(P // BLK,),
    in_specs=[_row_spec, _p_spec] + [_w_spec, _b_spec] * 5,
    out_specs=_row_spec,
    out_shape=jax.ShapeDtypeStruct((P, D), jnp.float32),
)


def kernel(x, edge_index, mp_w0, mp_b0, mp_w1, mp_b1, mp_w2, mp_b2,
           fc_w0, fc_b0, fc_w1, fc_b1, pj_w0, pj_b0, pj_w1, pj_b1,
           out_w, out_b):
    xp = jnp.pad(x, ((0, P - N_NODES), (0, 0)))
    ei3 = edge_index.reshape(2, N_EDGES // CH, CH)

    u = _u_call(xp, mp_w0, mp_b0.reshape(1, D))
    degp = _deg_call()(ei3)
    h, invn = _scale_call(u, degp)
    for w, b in ((mp_w1, mp_b1), (mp_w2, mp_b2)):
        part = _mp_call()(h, ei3)
        h = _combine_call(h, part, w, b.reshape(1, D), invn)
    part = _mp_call()(h, ei3)
    outp = _tail_call(h, part,
                      fc_w0, fc_b0.reshape(1, D), fc_w1, fc_b1.reshape(1, D),
                      pj_w0, pj_b0.reshape(1, D), pj_w1, pj_b1.reshape(1, D),
                      out_w, out_b.reshape(1, D))
    return outp[:N_NODES]
